# Initial kernel scaffold; baseline (speedup 1.0000x reference)
import jax
import jax.numpy as jnp
from jax.experimental import pallas as pl


def kernel(batch, table):
    raise NotImplementedError("write your pallas kernel here")


# speedup vs baseline: 1.8438x; 1.8438x over previous
import functools

import jax
import jax.numpy as jnp
from jax import lax
from jax.experimental import pallas as pl
from jax.experimental.pallas import tpu as pltpu
from jax.experimental.pallas import tpu_sc as plsc

_GROUP = 128
_K = 8
_NC = 2
_NS = 16
_NW = _NC * _NS


@functools.lru_cache(maxsize=None)
def _make_gather(vocab, n, emb_dim):
    assert n % (_NW * _K * _GROUP) == 0
    n_groups = n // _GROUP
    groups_per_w = n_groups // _NW
    n_super = groups_per_w // _K
    mesh = plsc.VectorSubcoreMesh(core_axis_name="c", subcore_axis_name="s")

    @functools.partial(
        pl.kernel,
        mesh=mesh,
        out_type=jax.ShapeDtypeStruct((n, emb_dim), jnp.float32),
        compiler_params=pltpu.CompilerParams(use_tc_tiling_on_sc=False),
        scratch_types=[
            pltpu.VMEM((_K, _GROUP), jnp.int32),
            pltpu.VMEM((_K * _GROUP, emb_dim), jnp.float32),
            pltpu.SemaphoreType.DMA,
        ],
    )
    def gather_kernel(idx_hbm, table_hbm, out_hbm, idx_v, rows_v, gsem):
        wid = lax.axis_index("s") * _NC + lax.axis_index("c")
        g_base = wid * groups_per_w

        def body(i, carry):
            g0 = g_base + i * _K
            pltpu.sync_copy(idx_hbm.at[pl.ds(g0, _K)], idx_v)
            copies = [
                pltpu.async_copy(
                    table_hbm.at[idx_v.at[j]],
                    rows_v.at[pl.ds(j * _GROUP, _GROUP)],
                    gsem,
                )
                for j in range(_K)
            ]
            for c in copies:
                c.wait()
            pltpu.sync_copy(rows_v, out_hbm.at[pl.ds(g0 * _GROUP, _K * _GROUP)])
            return carry

        lax.fori_loop(0, n_super, body, 0)

    return gather_kernel


def kernel(batch, table):
    b, h = batch.shape
    vocab, emb_dim = table.shape
    n = b * h
    idx2d = batch.reshape(n // _GROUP, _GROUP)
    out = _make_gather(vocab, n, emb_dim)(idx2d, table)
    return out.reshape(b, h, emb_dim)
